# baseline (speedup 1.0000x reference)
import functools

import jax
import jax.numpy as jnp
from jax import lax
from jax.experimental import pallas as pl
from jax.experimental.pallas import tpu as pltpu
from jax.experimental.pallas import tpu_sc as plsc

N = 4096
D = 128
NC, NS = 2, 16
NW = NC * NS
BLK = 512
CH = 128
NCHUNK = N // CH
NBUF = 2
NITER = NCHUNK // NBUF


def _rank_body(xrow_ref, xcol_ref, idx_ref):
    xrow = xrow_ref[...]
    xcol = xcol_ref[...]
    jrow = lax.broadcasted_iota(jnp.int32, (1, N), 1)

    acc = jnp.zeros((1, N), jnp.int32)
    for blk in range(N // BLK):
        a = xcol[blk * BLK:(blk + 1) * BLK, :]
        gi = lax.broadcasted_iota(jnp.int32, (BLK, 1), 0) + blk * BLK
        m = (a < xrow) | ((a == xrow) & (gi < jrow))
        acc = acc + jnp.sum(m.astype(jnp.int32), axis=0, keepdims=True)

    woff = lax.broadcasted_iota(jnp.int32, (NW, 1), 0) * N
    idx_ref[...] = acc + woff


def _rank_indices(x_flat):
    return pl.pallas_call(
        _rank_body,
        out_shape=jax.ShapeDtypeStruct((NW, N), jnp.int32),
    )(x_flat.reshape(1, N), x_flat.reshape(N, 1))


def _scatter_body(yflat, idxmat, out, idx_v, spbuf, buf0, buf1,
                  gsem0, gsem1, osem0, osem1):
    sid = lax.axis_index("s")
    w = sid * NC + lax.axis_index("c")
    pltpu.sync_copy(idxmat.at[w], idx_v)

    bufs = (buf0, buf1)
    gsems = (gsem0, gsem1)
    osems = (osem0, osem1)

    def fire_g(c, j):
        pltpu.async_copy(yflat.at[pl.ds(w * N + c * CH, CH)],
                         spbuf.at[sid].at[j], gsems[j])

    def wait_g(c, j):
        pltpu.make_async_copy(yflat.at[pl.ds(w * N + c * CH, CH)],
                              spbuf.at[sid].at[j], gsems[j]).wait()
        pltpu.sync_copy(spbuf.at[sid].at[j], bufs[j])

    def fire_o(c, j):
        pltpu.async_copy(bufs[j], out.at[idx_v.at[c]], osems[j])

    def wait_o(c, j):
        pltpu.make_async_copy(bufs[j], out.at[idx_v.at[c]], osems[j]).wait()

    for j in range(NBUF):
        fire_g(j, j)

    def body(i, _):
        c = i * NBUF
        for j in range(NBUF):
            wait_g(c + j, j)
            fire_o(c + j, j)
        for j in range(NBUF):
            wait_o(c + j, j)

            @pl.when(i < NITER - 1)
            def _():
                fire_g(c + NBUF + j, j)
        return 0

    lax.fori_loop(0, NITER, body, 0)


def _scatter_rows(yflat, idxmat):
    mesh = plsc.VectorSubcoreMesh(core_axis_name="c", subcore_axis_name="s")
    return pl.kernel(
        _scatter_body,
        out_type=jax.ShapeDtypeStruct((NW * N, D), jnp.float32),
        mesh=mesh,
        scratch_types=(
            [pltpu.VMEM((NCHUNK, CH), jnp.int32),
             pltpu.VMEM_SHARED((NS, NBUF, CH, D), jnp.float32)]
            + [pltpu.VMEM((CH, D), jnp.float32)] * NBUF
            + [pltpu.SemaphoreType.DMA] * (2 * NBUF)
        ),
    )(yflat, idxmat)


def kernel(x, y):
    idxmat = _rank_indices(x.reshape(N))
    yflat = y.reshape(NW * N, D)
    out = _scatter_rows(yflat, idxmat.reshape(NW, NCHUNK, CH))
    return out.reshape(y.shape)


# speedup vs baseline: 1.0572x; 1.0572x over previous
import functools

import jax
import jax.numpy as jnp
from jax import lax
from jax.experimental import pallas as pl
from jax.experimental.pallas import tpu as pltpu
from jax.experimental.pallas import tpu_sc as plsc

N = 4096
D = 128
NC, NS = 2, 16
NW = NC * NS
BLK = 512
CH = 128
NCHUNK = N // CH
NBUF = 4
NITER = NCHUNK // NBUF


def _rank_body(xrow_ref, idx_ref):
    xrow = xrow_ref[...]
    xcol = jnp.transpose(xrow)
    jrow = lax.broadcasted_iota(jnp.int32, (1, N), 1)

    acc = jnp.zeros((1, N), jnp.int32)
    for blk in range(N // BLK):
        a = xcol[blk * BLK:(blk + 1) * BLK, :]
        gi = lax.broadcasted_iota(jnp.int32, (BLK, 1), 0) + blk * BLK
        m = (a < xrow) | ((a == xrow) & (gi < jrow))
        acc = acc + jnp.sum(m.astype(jnp.int32), axis=0, keepdims=True)

    woff = lax.broadcasted_iota(jnp.int32, (NW, 1), 0) * N
    idx_ref[...] = acc + woff


def _rank_indices(x_flat):
    return pl.pallas_call(
        _rank_body,
        out_shape=jax.ShapeDtypeStruct((NW, N), jnp.int32),
    )(x_flat.reshape(1, N))


def _scatter_body(yflat, idxmat, out, idx_v, buf0, buf1, buf2, buf3,
                  gsem0, gsem1, gsem2, gsem3, osem0, osem1, osem2, osem3):
    w = lax.axis_index("s") * NC + lax.axis_index("c")
    pltpu.sync_copy(idxmat.at[w], idx_v)

    bufs = (buf0, buf1, buf2, buf3)
    gsems = (gsem0, gsem1, gsem2, gsem3)
    osems = (osem0, osem1, osem2, osem3)

    def fire_g(c, j):
        pltpu.async_copy(yflat.at[pl.ds(w * N + c * CH, CH)], bufs[j],
                         gsems[j])

    def wait_g(c, j):
        pltpu.make_async_copy(yflat.at[pl.ds(w * N + c * CH, CH)], bufs[j],
                              gsems[j]).wait()

    def fire_o(c, j):
        pltpu.async_copy(bufs[j], out.at[idx_v.at[c]], osems[j])

    def wait_o(c, j):
        pltpu.make_async_copy(bufs[j], out.at[idx_v.at[c]], osems[j]).wait()

    for j in range(NBUF):
        fire_g(j, j)

    def body(i, _):
        c = i * NBUF
        for j in range(NBUF):
            wait_g(c + j, j)
            fire_o(c + j, j)
        for j in range(NBUF):
            wait_o(c + j, j)

            @pl.when(i < NITER - 1)
            def _():
                fire_g(c + NBUF + j, j)
        return 0

    lax.fori_loop(0, NITER, body, 0)


def _scatter_rows(yflat, idxmat):
    mesh = plsc.VectorSubcoreMesh(core_axis_name="c", subcore_axis_name="s")
    return pl.kernel(
        _scatter_body,
        out_type=jax.ShapeDtypeStruct((NW * N, D), jnp.float32),
        mesh=mesh,
        scratch_types=(
            [pltpu.VMEM((NCHUNK, CH), jnp.int32)]
            + [pltpu.VMEM((CH, D), jnp.float32)] * NBUF
            + [pltpu.SemaphoreType.DMA] * (2 * NBUF)
        ),
    )(yflat, idxmat)


def kernel(x, y):
    idxmat = _rank_indices(x.reshape(N))
    yflat = y.reshape(NW * N, D)
    out = _scatter_rows(yflat, idxmat.reshape(NW, NCHUNK, CH))
    return out.reshape(y.shape)


# speedup vs baseline: 1.0911x; 1.0321x over previous
import functools

import jax
import jax.numpy as jnp
from jax import lax
from jax.experimental import pallas as pl
from jax.experimental.pallas import tpu as pltpu
from jax.experimental.pallas import tpu_sc as plsc

N = 4096
D = 128
NC, NS = 2, 16
NW = NC * NS
BLK = 512
CH = 128
NCHUNK = N // CH
NBUF = 4
NITER = NCHUNK // NBUF


def _rank_body(xrow_ref, idx_ref):
    xrow = xrow_ref[...]
    xcol = jnp.transpose(xrow)
    jrow = lax.broadcasted_iota(jnp.int32, (1, N), 1)

    acc = jnp.zeros((1, N), jnp.int32)
    for blk in range(N // BLK):
        a = xcol[blk * BLK:(blk + 1) * BLK, :]
        gi = lax.broadcasted_iota(jnp.int32, (BLK, 1), 0) + blk * BLK
        m = (a < xrow) | ((a == xrow) & (gi < jrow))
        acc = acc + jnp.sum(m.astype(jnp.int32), axis=0, keepdims=True)

    r = jnp.reshape(acc, (NCHUNK, CH))
    woff = lax.broadcasted_iota(jnp.int32, (NW, 1, 1), 0) * N
    idx_ref[...] = r[None, :, :] + woff


def _rank_indices(x_flat):
    return pl.pallas_call(
        _rank_body,
        out_shape=jax.ShapeDtypeStruct((NW, NCHUNK, CH), jnp.int32),
    )(x_flat.reshape(1, N))


def _scatter_body(yflat, idxmat, out, idx_v, buf0, buf1, buf2, buf3,
                  gsem0, gsem1, gsem2, gsem3, osem0, osem1, osem2, osem3):
    w = lax.axis_index("s") * NC + lax.axis_index("c")
    pltpu.sync_copy(idxmat.at[w], idx_v)

    bufs = (buf0, buf1, buf2, buf3)
    gsems = (gsem0, gsem1, gsem2, gsem3)
    osems = (osem0, osem1, osem2, osem3)

    def fire_g(c, j):
        pltpu.async_copy(yflat.at[pl.ds(w * N + c * CH, CH)], bufs[j],
                         gsems[j])

    def wait_g(c, j):
        pltpu.make_async_copy(yflat.at[pl.ds(w * N + c * CH, CH)], bufs[j],
                              gsems[j]).wait()

    def fire_o(c, j):
        pltpu.async_copy(bufs[j], out.at[idx_v.at[c]], osems[j])

    def wait_o(c, j):
        pltpu.make_async_copy(bufs[j], out.at[idx_v.at[c]], osems[j]).wait()

    for j in range(NBUF):
        fire_g(j, j)

    def body(i, _):
        c = i * NBUF
        for j in range(NBUF):
            wait_g(c + j, j)
            fire_o(c + j, j)
        for j in range(NBUF):
            wait_o(c + j, j)

            @pl.when(i < NITER - 1)
            def _():
                fire_g(c + NBUF + j, j)
        return 0

    lax.fori_loop(0, NITER, body, 0)


def _scatter_rows(yflat, idxmat):
    mesh = plsc.VectorSubcoreMesh(core_axis_name="c", subcore_axis_name="s")
    return pl.kernel(
        _scatter_body,
        out_type=jax.ShapeDtypeStruct((NW * N, D), jnp.float32),
        mesh=mesh,
        scratch_types=(
            [pltpu.VMEM((NCHUNK, CH), jnp.int32)]
            + [pltpu.VMEM((CH, D), jnp.float32)] * NBUF
            + [pltpu.SemaphoreType.DMA] * (2 * NBUF)
        ),
    )(yflat, idxmat)


def kernel(x, y):
    idxmat = _rank_indices(x.reshape(N))
    yflat = y.reshape(NW * N, D)
    out = _scatter_rows(yflat, idxmat.reshape(NW, NCHUNK, CH))
    return out.reshape(y.shape)


# speedup vs baseline: 1.1138x; 1.0209x over previous
import functools

import jax
import jax.numpy as jnp
from jax import lax
from jax.experimental import pallas as pl
from jax.experimental.pallas import tpu as pltpu
from jax.experimental.pallas import tpu_sc as plsc

N = 4096
D = 128
NC, NS = 2, 16
NW = NC * NS
BLK = 512
CH = 128
NCHUNK = N // CH
NBUF = 4
NITER = NCHUNK // NBUF


def _rank_body(xrow_ref, idx_ref):
    xrow = xrow_ref[...]
    xcol = jnp.transpose(xrow)
    jrow = lax.broadcasted_iota(jnp.int32, (1, N), 1)

    ones = jnp.ones((1, BLK), jnp.bfloat16)
    accf = jnp.zeros((1, N), jnp.float32)
    for blk in range(N // BLK):
        a = xcol[blk * BLK:(blk + 1) * BLK, :]
        gi = lax.broadcasted_iota(jnp.int32, (BLK, 1), 0) + blk * BLK
        m = (a < xrow) | ((a == xrow) & (gi < jrow))
        accf = accf + jax.lax.dot_general(
            ones, m.astype(jnp.bfloat16), (((1,), (0,)), ((), ())),
            preferred_element_type=jnp.float32)
    acc = accf.astype(jnp.int32)

    r = jnp.reshape(acc, (NCHUNK, CH))
    woff = lax.broadcasted_iota(jnp.int32, (NW, 1, 1), 0) * N
    idx_ref[...] = r[None, :, :] + woff


def _rank_indices(x_flat):
    return pl.pallas_call(
        _rank_body,
        out_shape=jax.ShapeDtypeStruct((NW, NCHUNK, CH), jnp.int32),
    )(x_flat.reshape(1, N))


def _scatter_body(yflat, idxmat, out, idx_v, buf0, buf1, buf2, buf3,
                  gsem0, gsem1, gsem2, gsem3, osem0, osem1, osem2, osem3):
    w = lax.axis_index("s") * NC + lax.axis_index("c")
    pltpu.sync_copy(idxmat.at[w], idx_v)

    bufs = (buf0, buf1, buf2, buf3)
    gsems = (gsem0, gsem1, gsem2, gsem3)
    osems = (osem0, osem1, osem2, osem3)

    def fire_g(c, j):
        pltpu.async_copy(yflat.at[pl.ds(w * N + c * CH, CH)], bufs[j],
                         gsems[j])

    def wait_g(c, j):
        pltpu.make_async_copy(yflat.at[pl.ds(w * N + c * CH, CH)], bufs[j],
                              gsems[j]).wait()

    def fire_o(c, j):
        pltpu.async_copy(bufs[j], out.at[idx_v.at[c]], osems[j])

    def wait_o(c, j):
        pltpu.make_async_copy(bufs[j], out.at[idx_v.at[c]], osems[j]).wait()

    for j in range(NBUF):
        fire_g(j, j)

    def body(i, _):
        c = i * NBUF
        for j in range(NBUF):
            wait_g(c + j, j)
            fire_o(c + j, j)
        for j in range(NBUF):
            wait_o(c + j, j)

            @pl.when(i < NITER - 1)
            def _():
                fire_g(c + NBUF + j, j)
        return 0

    lax.fori_loop(0, NITER, body, 0)


def _scatter_rows(yflat, idxmat):
    mesh = plsc.VectorSubcoreMesh(core_axis_name="c", subcore_axis_name="s")
    return pl.kernel(
        _scatter_body,
        out_type=jax.ShapeDtypeStruct((NW * N, D), jnp.float32),
        mesh=mesh,
        scratch_types=(
            [pltpu.VMEM((NCHUNK, CH), jnp.int32)]
            + [pltpu.VMEM((CH, D), jnp.float32)] * NBUF
            + [pltpu.SemaphoreType.DMA] * (2 * NBUF)
        ),
    )(yflat, idxmat)


def kernel(x, y):
    idxmat = _rank_indices(x.reshape(N))
    yflat = y.reshape(NW * N, D)
    out = _scatter_rows(yflat, idxmat.reshape(NW, NCHUNK, CH))
    return out.reshape(y.shape)


# speedup vs baseline: 1.1661x; 1.0469x over previous
import functools

import jax
import jax.numpy as jnp
from jax import lax
from jax.experimental import pallas as pl
from jax.experimental.pallas import tpu as pltpu
from jax.experimental.pallas import tpu_sc as plsc

N = 4096
D = 128
NC, NS = 2, 16
NW = NC * NS
BLK = 512
CH = 128
NCHUNK = N // CH
NBUF = 4
NITER = NCHUNK // NBUF


def _rank_body(xrow_ref, idx_ref):
    xrow = xrow_ref[...]
    xcol = jnp.transpose(xrow)
    jrow = lax.broadcasted_iota(jnp.int32, (1, N), 1)

    ones = jnp.ones((1, BLK), jnp.bfloat16)
    gi_l = lax.broadcasted_iota(jnp.int32, (BLK, BLK), 0)
    gj_l = lax.broadcasted_iota(jnp.int32, (BLK, BLK), 1)
    tri = gi_l < gj_l
    NB = N // BLK
    parts = []
    for bj in range(NB):
        xr = xrow[:, bj * BLK:(bj + 1) * BLK]
        accf = jnp.zeros((1, BLK), jnp.float32)
        for bi in range(NB):
            a = xcol[bi * BLK:(bi + 1) * BLK, :]
            if bi < bj:
                mb = a <= xr
            elif bi > bj:
                mb = a < xr
            else:
                mb = (a < xr) | ((a == xr) & tri)
            accf = accf + jax.lax.dot_general(
                ones, mb.astype(jnp.bfloat16), (((1,), (0,)), ((), ())),
                preferred_element_type=jnp.float32)
        parts.append(accf)
    acc = jnp.concatenate(parts, axis=1).astype(jnp.int32)

    r = jnp.reshape(acc, (NCHUNK, CH))
    woff = lax.broadcasted_iota(jnp.int32, (NW, 1, 1), 0) * N
    idx_ref[...] = r[None, :, :] + woff


def _rank_indices(x_flat):
    return pl.pallas_call(
        _rank_body,
        out_shape=jax.ShapeDtypeStruct((NW, NCHUNK, CH), jnp.int32),
    )(x_flat.reshape(1, N))


def _scatter_body(yflat, idxmat, out, idx_v, buf0, buf1, buf2, buf3,
                  gsem0, gsem1, gsem2, gsem3, osem0, osem1, osem2, osem3):
    w = lax.axis_index("s") * NC + lax.axis_index("c")
    pltpu.sync_copy(idxmat.at[w], idx_v)

    bufs = (buf0, buf1, buf2, buf3)
    gsems = (gsem0, gsem1, gsem2, gsem3)
    osems = (osem0, osem1, osem2, osem3)

    def fire_g(c, j):
        pltpu.async_copy(yflat.at[pl.ds(w * N + c * CH, CH)], bufs[j],
                         gsems[j])

    def wait_g(c, j):
        pltpu.make_async_copy(yflat.at[pl.ds(w * N + c * CH, CH)], bufs[j],
                              gsems[j]).wait()

    def fire_o(c, j):
        pltpu.async_copy(bufs[j], out.at[idx_v.at[c]], osems[j])

    def wait_o(c, j):
        pltpu.make_async_copy(bufs[j], out.at[idx_v.at[c]], osems[j]).wait()

    for j in range(NBUF):
        fire_g(j, j)

    def body(i, _):
        c = i * NBUF
        for j in range(NBUF):
            wait_g(c + j, j)
            fire_o(c + j, j)
        for j in range(NBUF):
            wait_o(c + j, j)

            @pl.when(i < NITER - 1)
            def _():
                fire_g(c + NBUF + j, j)
        return 0

    lax.fori_loop(0, NITER, body, 0)


def _scatter_rows(yflat, idxmat):
    mesh = plsc.VectorSubcoreMesh(core_axis_name="c", subcore_axis_name="s")
    return pl.kernel(
        _scatter_body,
        out_type=jax.ShapeDtypeStruct((NW * N, D), jnp.float32),
        mesh=mesh,
        scratch_types=(
            [pltpu.VMEM((NCHUNK, CH), jnp.int32)]
            + [pltpu.VMEM((CH, D), jnp.float32)] * NBUF
            + [pltpu.SemaphoreType.DMA] * (2 * NBUF)
        ),
    )(yflat, idxmat)


def kernel(x, y):
    idxmat = _rank_indices(x.reshape(N))
    yflat = y.reshape(NW * N, D)
    out = _scatter_rows(yflat, idxmat.reshape(NW, NCHUNK, CH))
    return out.reshape(y.shape)
